# Initial kernel scaffold; baseline (speedup 1.0000x reference)
import jax
import jax.numpy as jnp
from jax.experimental import pallas as pl


def kernel(X, Wt, bt, att_e, att_v, vertex, edges, v_type, e_idx0, e_idx1, e_idx2, e_idx3, e_inv, v_idx0, v_idx1, v_idx2, v_idx3, v_inv):
    raise NotImplementedError("write your pallas kernel here")


# speedup vs baseline: 22.7797x; 22.7797x over previous
import functools

import jax
import jax.numpy as jnp
from jax import lax
from jax.experimental import pallas as pl
from jax.experimental.pallas import tpu as pltpu
from jax.experimental.pallas import tpu_sc as plsc

N = 10000
NNZ = 320000
E = 20000
H = 8
C = 16
D = 128
HH = 64
NEG = 0.2

NT = 16
W = 80
PAIRS_PER_TILE = NNZ // NT
NWIN = PAIRS_PER_TILE // W
RCH = 80
NCH_A = E // RCH
NCH_C = N // RCH

MB = 1000


def _typed_linear_body(x_ref, vt_ref, wt_ref, bt_ref, o_ref):
    x = x_ref[...]
    vt = vt_ref[...]
    acc = jnp.zeros((MB, D), jnp.float32)
    for k in range(4):
        xk = jnp.dot(x, wt_ref[k], preferred_element_type=jnp.float32)
        xk = xk + bt_ref[k][None, :]
        acc = acc + jnp.where(vt == float(k), xk, 0.0)
    o_ref[0] = acc[:, :HH]
    o_ref[1] = acc[:, HH:]


def _typed_linear(x, vtf, wt, bt):
    return pl.pallas_call(
        _typed_linear_body,
        grid=(N // MB,),
        in_specs=[
            pl.BlockSpec((MB, D), lambda i: (i, 0)),
            pl.BlockSpec((MB, 1), lambda i: (i, 0)),
            pl.BlockSpec((4, D, H * C), lambda i: (0, 0, 0)),
            pl.BlockSpec((4, H * C), lambda i: (0, 0)),
        ],
        out_specs=pl.BlockSpec((2, MB, HH), lambda i: (0, i, 0)),
        out_shape=jax.ShapeDtypeStruct((2, N, HH), jnp.float32),
    )(x, vtf, wt, bt)


def _sc_mesh_kernel():
    mesh = plsc.VectorSubcoreMesh(core_axis_name="c", subcore_axis_name="s")

    @functools.partial(
        pl.kernel,
        out_type=(
            jax.ShapeDtypeStruct((2 * E, HH), jnp.float32),
            jax.ShapeDtypeStruct((2, N, HH), jnp.float32),
        ),
        mesh=mesh,
        compiler_params=pltpu.CompilerParams(use_tc_tiling_on_sc=False),
        scratch_types=[
            pltpu.VMEM_SHARED((E, HH), jnp.float32),
            pltpu.VMEM_SHARED((E, C), jnp.float32),
            pltpu.VMEM((W,), jnp.int32),
            pltpu.VMEM((W,), jnp.int32),
            pltpu.VMEM((W,), jnp.int32),
            pltpu.VMEM((W, HH), jnp.float32),
            pltpu.VMEM((W, HH), jnp.float32),
            pltpu.VMEM((W, C), jnp.float32),
            pltpu.VMEM((4 * HH,), jnp.float32),
            pltpu.VMEM((4 * HH,), jnp.float32),
            pltpu.VMEM((RCH, HH), jnp.float32),
            pltpu.VMEM((RCH, C), jnp.float32),
            pltpu.VMEM((RCH, HH), jnp.float32),
            pltpu.VMEM((RCH, C), jnp.float32),
            pltpu.SemaphoreType.DMA,
        ],
    )
    def body(xh2, vidx2, edges, eidx2, vertex, cls_e, cls_v, att_e2, att_v2,
             xe, out, acc_num, acc_den, gidx, sidx, clsb, rows, wrows, evals,
             att_eb, att_vb, nbuf, dbuf, zbuf, zdbuf, sem):
        sc = lax.axis_index("c")
        t = lax.axis_index("s")
        iota16 = lax.iota(jnp.int32, 16)
        zeros16 = jnp.zeros((16,), jnp.float32)

        gd = lax.GatherDimensionNumbers(
            offset_dims=(), collapsed_slice_dims=(0,), start_index_map=(0,))

        def lanesum(v):
            for k in (1, 2, 4, 8):
                shuf = lax.gather(v, (iota16 ^ k)[:, None], gd, (1,),
                                  mode=lax.GatherScatterMode.PROMISE_IN_BOUNDS)
                v = v + shuf
            return v

        def lanebcast(v, h):
            idx = jnp.full((16,), h, jnp.int32)
            return lax.gather(v, idx[:, None], gd, (1,),
                              mode=lax.GatherScatterMode.PROMISE_IN_BOUNDS)

        pltpu.sync_copy(att_e2.at[pl.ds(sc * (4 * HH), 4 * HH)], att_eb)
        pltpu.sync_copy(att_v2.at[pl.ds(sc * (4 * HH), 4 * HH)], att_vb)

        def zsrc(r, _):
            for j in range(HH // 16):
                zbuf[r, pl.ds(j * 16, 16)] = zeros16
            zdbuf[r, :] = zeros16
            return 0
        lax.fori_loop(0, RCH, zsrc, 0)

        def zero_region(nch):
            def zacc(i, _):
                cid = t + NT * i

                @pl.when(cid < nch)
                def _():
                    pltpu.sync_copy(zbuf, acc_num.at[pl.ds(cid * RCH, RCH)])
                    pltpu.sync_copy(zdbuf, acc_den.at[pl.ds(cid * RCH, RCH)])
                return 0
            lax.fori_loop(0, (nch + NT - 1) // NT, zacc, 0)

        def pair_pass(gsrc, ssrc, csrc, table, attT):
            def window(w, _):
                b = t * PAIRS_PER_TILE + w * W
                pltpu.sync_copy(gsrc.at[pl.ds(sc * NNZ + b, W)], gidx)
                pltpu.sync_copy(ssrc.at[pl.ds(b, W)], sidx)
                pltpu.sync_copy(csrc.at[pl.ds(b, W)], clsb)
                pltpu.async_copy(table.at[gidx], rows, sem).wait()

                def group(g, _):
                    base16 = g * 16
                    clsvec = clsb[pl.ds(base16, 16)]
                    for j in range(16):
                        p = base16 + j
                        cls = clsvec[j]
                        abase = cls * HH
                        easm = zeros16
                        for h in range(4):
                            r = rows[p, pl.ds(h * 16, 16)]
                            a = attT[pl.ds(abase + h * 16, 16)]
                            s = lanesum(r * a)
                            s = jnp.where(s > 0, s, NEG * s)
                            evh = jnp.exp(s)
                            wrows[p, pl.ds(h * 16, 16)] = r * evh
                            easm = jnp.where(iota16 == h, evh, easm)
                        evals[p, :] = easm
                    return 0
                lax.fori_loop(0, W // 16, group, 0)

                pltpu.sync_copy(wrows, acc_num.at[sidx], add=True)
                pltpu.sync_copy(evals, acc_den.at[sidx], add=True)
                return 0
            lax.fori_loop(0, NWIN, window, 0)

        def normalize(nch, dst_hbm, dst_base, do_relu):
            def chunk(i, _):
                cid = t + NT * i

                @pl.when(cid < nch)
                def _():
                    rbase = cid * RCH
                    pltpu.sync_copy(acc_num.at[pl.ds(rbase, RCH)], nbuf)
                    pltpu.sync_copy(acc_den.at[pl.ds(rbase, RCH)], dbuf)

                    def row(r, _):
                        recd = 1.0 / (dbuf[r, :] + 1e-16)
                        for h in range(4):
                            v = nbuf[r, pl.ds(h * 16, 16)] * lanebcast(recd, h)
                            if do_relu:
                                v = jnp.maximum(v, 0.0)
                            nbuf[r, pl.ds(h * 16, 16)] = v
                        return 0
                    lax.fori_loop(0, RCH, row, 0)
                    pltpu.sync_copy(nbuf, dst_hbm.at[pl.ds(dst_base + rbase, RCH)])
                return 0
            lax.fori_loop(0, (nch + NT - 1) // NT, chunk, 0)

        zero_region(NCH_A)
        plsc.subcore_barrier()
        pair_pass(vidx2, edges, cls_e, xh2, att_eb)
        plsc.subcore_barrier()
        normalize(NCH_A, xe, sc * E, True)
        plsc.subcore_barrier()

        zero_region(NCH_C)
        plsc.subcore_barrier()
        pair_pass(eidx2, vertex, cls_v, xe, att_vb)
        plsc.subcore_barrier()

        def out_chunk(i, _):
            cid = t + NT * i

            @pl.when(cid < NCH_C)
            def _():
                rbase = cid * RCH
                pltpu.sync_copy(acc_num.at[pl.ds(rbase, RCH)], nbuf)
                pltpu.sync_copy(acc_den.at[pl.ds(rbase, RCH)], dbuf)

                def row(r, _):
                    recd = 1.0 / (dbuf[r, :] + 1e-16)
                    for h in range(4):
                        nbuf[r, pl.ds(h * 16, 16)] = (
                            nbuf[r, pl.ds(h * 16, 16)] * lanebcast(recd, h))
                    return 0
                lax.fori_loop(0, RCH, row, 0)
                pltpu.sync_copy(nbuf, out.at[sc, pl.ds(rbase, RCH)])
            return 0
        lax.fori_loop(0, (NCH_C + NT - 1) // NT, out_chunk, 0)

    return body


_sc_kernel = _sc_mesh_kernel()


def kernel(X, Wt, bt, att_e, att_v, vertex, edges, v_type,
           e_idx0, e_idx1, e_idx2, e_idx3, e_inv,
           v_idx0, v_idx1, v_idx2, v_idx3, v_inv):
    vertex = vertex.astype(jnp.int32)
    edges = edges.astype(jnp.int32)

    se0 = e_idx0.shape[0]
    se1 = se0 + e_idx1.shape[0]
    se2 = se1 + e_idx2.shape[0]
    cls_e = ((e_inv >= se0).astype(jnp.int32) + (e_inv >= se1)
             + (e_inv >= se2))
    sv0 = v_idx0.shape[0]
    sv1 = sv0 + v_idx1.shape[0]
    sv2 = sv1 + v_idx2.shape[0]
    cls_v = ((v_inv >= sv0).astype(jnp.int32) + (v_inv >= sv1)
             + (v_inv >= sv2))

    vidx2 = jnp.concatenate([vertex, vertex + N])
    eidx2 = jnp.concatenate([edges, edges + E])

    ae = att_e.reshape(4, H * C)
    av = att_v.reshape(4, H * C)
    att_e2 = jnp.concatenate([ae[:, :HH].reshape(-1), ae[:, HH:].reshape(-1)])
    att_v2 = jnp.concatenate([av[:, :HH].reshape(-1), av[:, HH:].reshape(-1)])

    vtf = v_type.astype(jnp.float32).reshape(N, 1)
    xh = _typed_linear(X, vtf, Wt, bt)
    xh2 = xh.reshape(2 * N, HH)

    _, out = _sc_kernel(xh2, vidx2, edges, eidx2, vertex,
                        cls_e, cls_v, att_e2, att_v2)
    return out.transpose(1, 0, 2).reshape(N, H * C)


# speedup vs baseline: 45.6848x; 2.0055x over previous
import functools

import jax
import jax.numpy as jnp
from jax import lax
from jax.experimental import pallas as pl
from jax.experimental.pallas import tpu as pltpu
from jax.experimental.pallas import tpu_sc as plsc

N = 10000
NNZ = 320000
E = 20000
H = 8
C = 16
D = 128
HH = 64
TW = 80
NEG = 0.2

NT = 16
W = 80
PAIRS_PER_TILE = NNZ // NT
NWIN = PAIRS_PER_TILE // W
RCH = 80

MB = 1000


def _typed_linear_body(x_ref, vt_ref, wt_ref, bt_ref, me_ref, o_ref):
    x = x_ref[...]
    vt = vt_ref[...]
    acc = jnp.zeros((MB, D), jnp.float32)
    for k in range(4):
        xk = jnp.dot(x, wt_ref[k], preferred_element_type=jnp.float32)
        xk = xk + bt_ref[k][None, :]
        acc = acc + jnp.where(vt == float(k), xk, 0.0)
    stab = jnp.dot(acc, me_ref[...], preferred_element_type=jnp.float32)
    o_ref[0] = jnp.concatenate([acc[:, :HH], stab[:, :16]], axis=1)
    o_ref[1] = jnp.concatenate([acc[:, HH:], stab[:, 16:]], axis=1)


def _typed_linear(x, vtf, wt, bt, me):
    return pl.pallas_call(
        _typed_linear_body,
        grid=(N // MB,),
        in_specs=[
            pl.BlockSpec((MB, D), lambda i: (i, 0)),
            pl.BlockSpec((MB, 1), lambda i: (i, 0)),
            pl.BlockSpec((4, D, H * C), lambda i: (0, 0, 0)),
            pl.BlockSpec((4, H * C), lambda i: (0, 0)),
            pl.BlockSpec((D, 32), lambda i: (0, 0)),
        ],
        out_specs=pl.BlockSpec((2, MB, TW), lambda i: (0, i, 0)),
        out_shape=jax.ShapeDtypeStruct((2, N, TW), jnp.float32),
    )(x, vtf, wt, bt, me)


def _edge_score_body(xe_ref, mv_ref, o_ref):
    o_ref[...] = jnp.dot(xe_ref[...], mv_ref[0],
                         preferred_element_type=jnp.float32)


def _edge_scores(xe, mv):
    return pl.pallas_call(
        _edge_score_body,
        grid=(2 * E // MB,),
        in_specs=[
            pl.BlockSpec((MB, HH), lambda i: (i, 0)),
            pl.BlockSpec((1, HH, C), lambda i: (i // (E // MB), 0, 0)),
        ],
        out_specs=pl.BlockSpec((MB, C), lambda i: (i, 0)),
        out_shape=jax.ShapeDtypeStruct((2 * E, C), jnp.float32),
    )(xe, mv)


def _sc_round_kernel(first):
    NSEG = E if first else N
    NCH = NSEG // RCH
    mesh = plsc.VectorSubcoreMesh(core_axis_name="c", subcore_axis_name="s")
    if first:
        out_type = jax.ShapeDtypeStruct((2 * E, HH), jnp.float32)
    else:
        out_type = jax.ShapeDtypeStruct((2, N, HH), jnp.float32)

    scratch = [
        pltpu.VMEM_SHARED((NSEG, HH), jnp.float32),
        pltpu.VMEM_SHARED((NSEG, C), jnp.float32),
        pltpu.VMEM((W,), jnp.int32),
        pltpu.VMEM((W,), jnp.int32),
        pltpu.VMEM((W,), jnp.int32),
        pltpu.VMEM((W, TW if first else HH), jnp.float32),
        pltpu.VMEM((W, C), jnp.float32),
        pltpu.VMEM((W, HH), jnp.float32),
        pltpu.VMEM((W, C), jnp.float32),
        pltpu.VMEM((RCH, HH), jnp.float32),
        pltpu.VMEM((RCH, C), jnp.float32),
        pltpu.VMEM((RCH, HH), jnp.float32),
        pltpu.VMEM((RCH, C), jnp.float32),
        pltpu.SemaphoreType.DMA,
        pltpu.SemaphoreType.DMA,
    ]

    @functools.partial(
        pl.kernel,
        out_type=out_type,
        mesh=mesh,
        compiler_params=pltpu.CompilerParams(use_tc_tiling_on_sc=False),
        scratch_types=scratch,
    )
    def body(table, stable, gsrc, ssrc, csrc, out, acc_num, acc_den,
             gidx, sidx, clsb, rows, srows, wrows, evals,
             nbuf, dbuf, zbuf, zdbuf, sem, sem2):
        sc = lax.axis_index("c")
        t = lax.axis_index("s")
        iota16 = lax.iota(jnp.int32, 16)
        zeros16 = jnp.zeros((16,), jnp.float32)
        quad = iota16 & 3

        gd = lax.GatherDimensionNumbers(
            offset_dims=(), collapsed_slice_dims=(0,), start_index_map=(0,))

        def vshuf(v, idx):
            return lax.gather(v, idx[:, None], gd, (1,),
                              mode=lax.GatherScatterMode.PROMISE_IN_BOUNDS)

        def zsrc(r, _):
            for j in range(HH // 16):
                zbuf[r, pl.ds(j * 16, 16)] = zeros16
            zdbuf[r, :] = zeros16
            return 0
        lax.fori_loop(0, RCH, zsrc, 0)

        def zacc(i, _):
            cid = t + NT * i

            @pl.when(cid < NCH)
            def _():
                pltpu.sync_copy(zbuf, acc_num.at[pl.ds(cid * RCH, RCH)])
                pltpu.sync_copy(zdbuf, acc_den.at[pl.ds(cid * RCH, RCH)])
            return 0
        lax.fori_loop(0, (NCH + NT - 1) // NT, zacc, 0)
        plsc.subcore_barrier()

        def window(w, _):
            b = t * PAIRS_PER_TILE + w * W
            pltpu.sync_copy(gsrc.at[pl.ds(sc * NNZ + b, W)], gidx)
            pltpu.sync_copy(ssrc.at[pl.ds(b, W)], sidx)
            pltpu.sync_copy(csrc.at[pl.ds(b, W)], clsb)
            cp1 = pltpu.async_copy(table.at[gidx], rows, sem)
            if not first:
                cp2 = pltpu.async_copy(stable.at[gidx], srows, sem2)
            cp1.wait()
            if not first:
                cp2.wait()

            def group(g, _):
                base16 = g * 16
                clsvec = clsb[pl.ds(base16, 16)]
                for j in range(16):
                    p = base16 + j
                    cls = clsvec[j]
                    if first:
                        svec = rows[p, pl.ds(HH, 16)]
                    else:
                        svec = srows[p, :]
                    svec = jnp.where(svec > 0, svec, NEG * svec)
                    ev = jnp.exp(svec)
                    for h in range(4):
                        evh = vshuf(ev, jnp.full((16,), cls * 4 + h,
                                                 jnp.int32))
                        r = rows[p, pl.ds(h * 16, 16)]
                        wrows[p, pl.ds(h * 16, 16)] = r * evh
                    easm = vshuf(ev, cls * 4 + quad)
                    evals[p, :] = jnp.where(iota16 < 4, easm, 0.0)
                return 0
            lax.fori_loop(0, W // 16, group, 0)

            pltpu.sync_copy(wrows, acc_num.at[sidx], add=True)
            pltpu.sync_copy(evals, acc_den.at[sidx], add=True)
            return 0
        lax.fori_loop(0, NWIN, window, 0)
        plsc.subcore_barrier()

        def chunk(i, _):
            cid = t + NT * i

            @pl.when(cid < NCH)
            def _():
                rbase = cid * RCH
                pltpu.sync_copy(acc_num.at[pl.ds(rbase, RCH)], nbuf)
                pltpu.sync_copy(acc_den.at[pl.ds(rbase, RCH)], dbuf)

                def row(r, _):
                    recd = 1.0 / (dbuf[r, :] + 1e-16)
                    for h in range(4):
                        v = nbuf[r, pl.ds(h * 16, 16)] * vshuf(
                            recd, jnp.full((16,), h, jnp.int32))
                        if first:
                            v = jnp.maximum(v, 0.0)
                        nbuf[r, pl.ds(h * 16, 16)] = v
                    return 0
                lax.fori_loop(0, RCH, row, 0)
                if first:
                    pltpu.sync_copy(nbuf, out.at[pl.ds(sc * E + rbase, RCH)])
                else:
                    pltpu.sync_copy(nbuf, out.at[sc, pl.ds(rbase, RCH)])
            return 0
        lax.fori_loop(0, (NCH + NT - 1) // NT, chunk, 0)

    return body


_sc_round1 = _sc_round_kernel(True)
_sc_round2 = _sc_round_kernel(False)


def kernel(X, Wt, bt, att_e, att_v, vertex, edges, v_type,
           e_idx0, e_idx1, e_idx2, e_idx3, e_inv,
           v_idx0, v_idx1, v_idx2, v_idx3, v_inv):
    vertex = vertex.astype(jnp.int32)
    edges = edges.astype(jnp.int32)

    se0 = e_idx0.shape[0]
    se1 = se0 + e_idx1.shape[0]
    se2 = se1 + e_idx2.shape[0]
    cls_e = ((e_inv >= se0).astype(jnp.int32) + (e_inv >= se1)
             + (e_inv >= se2))
    sv0 = v_idx0.shape[0]
    sv1 = sv0 + v_idx1.shape[0]
    sv2 = sv1 + v_idx2.shape[0]
    cls_v = ((v_inv >= sv0).astype(jnp.int32) + (v_inv >= sv1)
             + (v_inv >= sv2))

    vidx2 = jnp.concatenate([vertex, vertex + N])
    eidx2 = jnp.concatenate([edges, edges + E])

    ae = att_e.reshape(4, H, C)
    eye8 = jnp.eye(H, dtype=jnp.float32).reshape(H, 2, 4)
    me = (ae.transpose(1, 2, 0)[:, :, None, :, None]
          * eye8[:, None, :, None, :])
    me = me.reshape(D, 32)

    av = att_v.reshape(4, H, C).transpose(2, 0, 1).reshape(C, 4, 2, 4)
    eye4 = jnp.eye(4, dtype=jnp.float32)
    mv = (av.transpose(2, 3, 0, 1)[:, :, :, :, None]
          * eye4[None, :, None, None, :])
    mv = mv.reshape(2, HH, C)

    vtf = v_type.astype(jnp.float32).reshape(N, 1)
    xh = _typed_linear(X, vtf, Wt, bt, me)
    xh2 = xh.reshape(2 * N, TW)

    xe = _sc_round1(xh2, xh2, vidx2, edges, cls_e)
    xes = _edge_scores(xe, mv)
    out = _sc_round2(xe, xes, eidx2, vertex, cls_v)
    return out.transpose(1, 0, 2).reshape(N, H * C)


# speedup vs baseline: 140.4437x; 3.0742x over previous
import functools

import jax
import jax.numpy as jnp
from jax import lax
from jax.experimental import pallas as pl
from jax.experimental.pallas import tpu as pltpu
from jax.experimental.pallas import tpu_sc as plsc

N = 10000
NNZ = 320000
E = 20000
H = 8
C = 16
D = 128
HH = 64
TW = 80
NEG = 0.2

NT = 16
W = 80
PAIRS_PER_TILE = NNZ // NT
NWIN = PAIRS_PER_TILE // W
RCH = 40

MB = 1000


def _typed_linear_body(x_ref, vt_ref, wt_ref, bt_ref, me_ref, o_ref):
    x = x_ref[...]
    vt = vt_ref[...]
    acc = jnp.zeros((MB, D), jnp.float32)
    for k in range(4):
        xk = jnp.dot(x, wt_ref[k], preferred_element_type=jnp.float32)
        xk = xk + bt_ref[k][None, :]
        acc = acc + jnp.where(vt == float(k), xk, 0.0)
    stab = jnp.dot(acc, me_ref[...], preferred_element_type=jnp.float32)
    o_ref[0] = jnp.concatenate([acc[:, :HH], stab[:, :16]], axis=1)
    o_ref[1] = jnp.concatenate([acc[:, HH:], stab[:, 16:]], axis=1)


def _typed_linear(x, vtf, wt, bt, me):
    return pl.pallas_call(
        _typed_linear_body,
        grid=(N // MB,),
        in_specs=[
            pl.BlockSpec((MB, D), lambda i: (i, 0)),
            pl.BlockSpec((MB, 1), lambda i: (i, 0)),
            pl.BlockSpec((4, D, H * C), lambda i: (0, 0, 0)),
            pl.BlockSpec((4, H * C), lambda i: (0, 0)),
            pl.BlockSpec((D, 32), lambda i: (0, 0)),
        ],
        out_specs=pl.BlockSpec((2, MB, TW), lambda i: (0, i, 0)),
        out_shape=jax.ShapeDtypeStruct((2, N, TW), jnp.float32),
    )(x, vtf, wt, bt, me)


def _edge_score_body(xe_ref, mv_ref, o_ref):
    feats = xe_ref[:, :HH]
    stab = jnp.dot(feats, mv_ref[0], preferred_element_type=jnp.float32)
    o_ref[...] = jnp.concatenate([feats, stab], axis=1)


def _edge_scores(xe, mv):
    return pl.pallas_call(
        _edge_score_body,
        grid=(2 * E // MB,),
        in_specs=[
            pl.BlockSpec((MB, TW), lambda i: (i, 0)),
            pl.BlockSpec((1, HH, C), lambda i: (i // (E // MB), 0, 0)),
        ],
        out_specs=pl.BlockSpec((MB, TW), lambda i: (i, 0)),
        out_shape=jax.ShapeDtypeStruct((2 * E, TW), jnp.float32),
    )(xe, mv)


def _sc_round_kernel(first):
    NSEG = E if first else N
    NCH = NSEG // RCH
    mesh = plsc.VectorSubcoreMesh(core_axis_name="c", subcore_axis_name="s")
    if first:
        out_type = jax.ShapeDtypeStruct((2 * E, TW), jnp.float32)
    else:
        out_type = jax.ShapeDtypeStruct((2, N, TW), jnp.float32)

    scratch = [
        pltpu.VMEM_SHARED((NSEG, TW), jnp.float32),
        pltpu.VMEM((RCH, TW), jnp.float32),
    ]
    for _ in range(2):
        scratch += [
            pltpu.VMEM((W,), jnp.int32),
            pltpu.VMEM((W,), jnp.int32),
            pltpu.VMEM((W,), jnp.int32),
            pltpu.VMEM((W, TW), jnp.float32),
            pltpu.VMEM((W, TW), jnp.float32),
            pltpu.SemaphoreType.DMA,
            pltpu.SemaphoreType.DMA,
            pltpu.SemaphoreType.DMA,
        ]

    @functools.partial(
        pl.kernel,
        out_type=out_type,
        mesh=mesh,
        compiler_params=pltpu.CompilerParams(use_tc_tiling_on_sc=False),
        scratch_types=scratch,
    )
    def body(table, gsrc, ssrc, csrc, out, acc, nbuf, *bufs):
        sc = lax.axis_index("c")
        t = lax.axis_index("s")
        iota16 = lax.iota(jnp.int32, 16)
        zeros16 = jnp.zeros((16,), jnp.float32)
        quad = iota16 & 3

        gd = lax.GatherDimensionNumbers(
            offset_dims=(), collapsed_slice_dims=(0,), start_index_map=(0,))

        def vshuf(v, idx):
            return lax.gather(v, idx[:, None], gd, (1,),
                              mode=lax.GatherScatterMode.PROMISE_IN_BOUNDS)

        def zsrc(r, _):
            for j in range(TW // 16):
                nbuf[r, pl.ds(j * 16, 16)] = zeros16
            return 0
        lax.fori_loop(0, RCH, zsrc, 0)

        def zacc(i, _):
            cid = t + NT * i

            @pl.when(cid < NCH)
            def _():
                pltpu.sync_copy(nbuf, acc.at[pl.ds(cid * RCH, RCH)])
            return 0
        lax.fori_loop(0, (NCH + NT - 1) // NT, zacc, 0)
        plsc.subcore_barrier()

        par = [dict(zip(
            ["gidx", "sidx", "clsb", "rows", "wrows",
             "semI", "semG", "semS"],
            bufs[i * 8:(i + 1) * 8])) for i in range(2)]

        def issue_gidx(P, w):
            b = sc * NNZ + t * PAIRS_PER_TILE + w * W
            pltpu.async_copy(gsrc.at[pl.ds(b, W)], P["gidx"], P["semI"])

        def wait_gidx(P):
            pltpu.make_async_copy(
                gsrc.at[pl.ds(0, W)], P["gidx"], P["semI"]).wait()

        def issue_gather(P):
            pltpu.async_copy(table.at[P["gidx"]], P["rows"], P["semG"])

        def wait_gather(P):
            pltpu.make_async_copy(
                table.at[P["gidx"]], P["rows"], P["semG"]).wait()

        def issue_scatter(P):
            pltpu.async_copy(P["wrows"], acc.at[P["sidx"]], P["semS"],
                             add=True)

        def wait_scatter(P):
            pltpu.make_async_copy(P["wrows"], acc.at[P["sidx"]],
                                  P["semS"]).wait()

        def compute(P, w):
            b = t * PAIRS_PER_TILE + w * W
            pltpu.sync_copy(ssrc.at[pl.ds(b, W)], P["sidx"])
            pltpu.sync_copy(csrc.at[pl.ds(b, W)], P["clsb"])
            rows, wrows = P["rows"], P["wrows"]

            def group(g, _):
                base16 = g * 16
                clsvec = P["clsb"][pl.ds(base16, 16)]
                for j in range(16):
                    p = base16 + j
                    cls = clsvec[j]
                    svec = rows[p, pl.ds(HH, 16)]
                    svec = jnp.where(svec > 0, svec, NEG * svec)
                    ev = jnp.exp(svec)
                    for h in range(4):
                        evh = vshuf(ev, jnp.full((16,), cls * 4 + h,
                                                 jnp.int32))
                        r = rows[p, pl.ds(h * 16, 16)]
                        wrows[p, pl.ds(h * 16, 16)] = r * evh
                    easm = vshuf(ev, cls * 4 + quad)
                    wrows[p, pl.ds(HH, 16)] = jnp.where(iota16 < 4, easm,
                                                        0.0)
                return 0
            lax.fori_loop(0, W // 16, group, 0)

        A, B = par
        if True:
            issue_gidx(A, 0)
            issue_gidx(B, 1)
            wait_gidx(A)
            issue_gather(A)

            def pipe(i, _):
                w0 = 2 * i
                wait_gather(A)
                issue_gidx(A, w0 + 2)
                wait_gidx(B)
                issue_gather(B)

                @pl.when(i > 0)
                def _():
                    wait_scatter(A)
                compute(A, w0)
                issue_scatter(A)

                w1 = w0 + 1
                wait_gather(B)
                issue_gidx(B, w1 + 2)

                @pl.when(w1 + 2 <= NWIN)
                def _():
                    wait_gidx(A)
                    issue_gather(A)

                @pl.when(i > 0)
                def _():
                    wait_scatter(B)
                compute(B, w1)
                issue_scatter(B)
                return 0
            lax.fori_loop(0, NWIN // 2, pipe, 0)
            wait_gidx(A)
            wait_gidx(B)
            wait_scatter(A)
            wait_scatter(B)
            plsc.subcore_barrier()

        def chunk(i, _):
            cid = t + NT * i

            @pl.when(cid < NCH)
            def _():
                rbase = cid * RCH
                pltpu.sync_copy(acc.at[pl.ds(rbase, RCH)], nbuf)

                def row(r, _):
                    recd = 1.0 / (nbuf[r, pl.ds(HH, 16)] + 1e-16)
                    for h in range(4):
                        v = nbuf[r, pl.ds(h * 16, 16)] * vshuf(
                            recd, jnp.full((16,), h, jnp.int32))
                        if first:
                            v = jnp.maximum(v, 0.0)
                        nbuf[r, pl.ds(h * 16, 16)] = v
                    return 0
                lax.fori_loop(0, RCH, row, 0)
                if first:
                    pltpu.sync_copy(nbuf, out.at[pl.ds(sc * E + rbase, RCH)])
                else:
                    pltpu.sync_copy(nbuf, out.at[sc, pl.ds(rbase, RCH)])
            return 0
        lax.fori_loop(0, (NCH + NT - 1) // NT, chunk, 0)

    return body


_sc_round1 = _sc_round_kernel(True)
_sc_round2 = _sc_round_kernel(False)


def kernel(X, Wt, bt, att_e, att_v, vertex, edges, v_type,
           e_idx0, e_idx1, e_idx2, e_idx3, e_inv,
           v_idx0, v_idx1, v_idx2, v_idx3, v_inv):
    vertex = vertex.astype(jnp.int32)
    edges = edges.astype(jnp.int32)

    se0 = e_idx0.shape[0]
    se1 = se0 + e_idx1.shape[0]
    se2 = se1 + e_idx2.shape[0]
    cls_e = ((e_inv >= se0).astype(jnp.int32) + (e_inv >= se1)
             + (e_inv >= se2))
    sv0 = v_idx0.shape[0]
    sv1 = sv0 + v_idx1.shape[0]
    sv2 = sv1 + v_idx2.shape[0]
    cls_v = ((v_inv >= sv0).astype(jnp.int32) + (v_inv >= sv1)
             + (v_inv >= sv2))

    pad = jnp.zeros((2 * W,), jnp.int32)
    vidx2 = jnp.concatenate([vertex, vertex + N, pad])
    eidx2 = jnp.concatenate([edges, edges + E, pad])

    ae = att_e.reshape(4, H, C)
    eye8 = jnp.eye(H, dtype=jnp.float32).reshape(H, 2, 4)
    me = (ae.transpose(1, 2, 0)[:, :, None, :, None]
          * eye8[:, None, :, None, :])
    me = me.reshape(D, 32)

    av = att_v.reshape(4, H, C).transpose(2, 0, 1).reshape(C, 4, 2, 4)
    eye4 = jnp.eye(4, dtype=jnp.float32)
    mv = (av.transpose(2, 3, 0, 1)[:, :, :, :, None]
          * eye4[None, :, None, None, :])
    mv = mv.reshape(2, HH, C)

    vtf = v_type.astype(jnp.float32).reshape(N, 1)
    xh = _typed_linear(X, vtf, Wt, bt, me)
    xh2 = xh.reshape(2 * N, TW)

    xe = _sc_round1(xh2, vidx2, edges, cls_e)
    xe2 = _edge_scores(xe, mv)
    out = _sc_round2(xe2, eidx2, vertex, cls_v)
    return out[:, :, :HH].transpose(1, 0, 2).reshape(N, H * C)


# speedup vs baseline: 160.8610x; 1.1454x over previous
import functools

import jax
import jax.numpy as jnp
from jax import lax
from jax.experimental import pallas as pl
from jax.experimental.pallas import tpu as pltpu
from jax.experimental.pallas import tpu_sc as plsc

N = 10000
NNZ = 320000
E = 20000
H = 8
C = 16
D = 128
HH = 64
TW = 80
NEG = 0.2

NT = 16
W = 80
PAIRS_PER_TILE = NNZ // NT
NWIN = PAIRS_PER_TILE // W
RCH = 40

MB = 1000


def _typed_linear_body(x_ref, vt_ref, wt_ref, bt_ref, me_ref, o_ref):
    x = x_ref[...]
    vt = vt_ref[...]
    acc = jnp.zeros((MB, D), jnp.float32)
    for k in range(4):
        xk = jnp.dot(x, wt_ref[k], preferred_element_type=jnp.float32)
        xk = xk + bt_ref[k][None, :]
        acc = acc + jnp.where(vt == float(k), xk, 0.0)
    stab = jnp.dot(acc, me_ref[...], preferred_element_type=jnp.float32)
    o_ref[0] = jnp.concatenate([acc[:, :HH], stab[:, :16]], axis=1)
    o_ref[1] = jnp.concatenate([acc[:, HH:], stab[:, 16:]], axis=1)


def _typed_linear(x, vtf, wt, bt, me):
    return pl.pallas_call(
        _typed_linear_body,
        grid=(N // MB,),
        in_specs=[
            pl.BlockSpec((MB, D), lambda i: (i, 0)),
            pl.BlockSpec((MB, 1), lambda i: (i, 0)),
            pl.BlockSpec((4, D, H * C), lambda i: (0, 0, 0)),
            pl.BlockSpec((4, H * C), lambda i: (0, 0)),
            pl.BlockSpec((D, 32), lambda i: (0, 0)),
        ],
        out_specs=pl.BlockSpec((2, MB, TW), lambda i: (0, i, 0)),
        out_shape=jax.ShapeDtypeStruct((2, N, TW), jnp.float32),
    )(x, vtf, wt, bt, me)


def _edge_score_body(xe_ref, mv_ref, o_ref):
    feats = xe_ref[:, :HH]
    stab = jnp.dot(feats, mv_ref[0], preferred_element_type=jnp.float32)
    o_ref[...] = jnp.concatenate([feats, stab], axis=1)


def _edge_scores(xe, mv):
    return pl.pallas_call(
        _edge_score_body,
        grid=(2 * E // MB,),
        in_specs=[
            pl.BlockSpec((MB, TW), lambda i: (i, 0)),
            pl.BlockSpec((1, HH, C), lambda i: (i // (E // MB), 0, 0)),
        ],
        out_specs=pl.BlockSpec((MB, TW), lambda i: (i, 0)),
        out_shape=jax.ShapeDtypeStruct((2 * E, TW), jnp.float32),
    )(xe, mv)


def _sc_round_kernel(first):
    NSEG = E if first else N
    NCH = NSEG // RCH
    mesh = plsc.VectorSubcoreMesh(core_axis_name="c", subcore_axis_name="s")
    if first:
        out_type = jax.ShapeDtypeStruct((2 * E, TW), jnp.float32)
    else:
        out_type = jax.ShapeDtypeStruct((2, N, TW), jnp.float32)

    scratch = [
        pltpu.VMEM_SHARED((NSEG, TW), jnp.float32),
        pltpu.VMEM((RCH, TW), jnp.float32),
    ]
    for _ in range(2):
        scratch += [
            pltpu.VMEM((2, W), jnp.int32),
            pltpu.VMEM((W,), jnp.int32),
            pltpu.VMEM((W, TW), jnp.float32),
            pltpu.VMEM((W, TW), jnp.float32),
            pltpu.SemaphoreType.DMA,
            pltpu.SemaphoreType.DMA,
            pltpu.SemaphoreType.DMA,
            pltpu.SemaphoreType.DMA,
        ]

    @functools.partial(
        pl.kernel,
        out_type=out_type,
        mesh=mesh,
        compiler_params=pltpu.CompilerParams(use_tc_tiling_on_sc=False),
        scratch_types=scratch,
    )
    def body(table, gsrc, ssrc, out, acc, nbuf, *bufs):
        sc = lax.axis_index("c")
        t = lax.axis_index("s")
        iota16 = lax.iota(jnp.int32, 16)
        zeros16 = jnp.zeros((16,), jnp.float32)
        quad = iota16 & 3

        gd = lax.GatherDimensionNumbers(
            offset_dims=(), collapsed_slice_dims=(0,), start_index_map=(0,))

        def vshuf(v, idx):
            return lax.gather(v, idx[:, None], gd, (1,),
                              mode=lax.GatherScatterMode.PROMISE_IN_BOUNDS)

        def zsrc(r, _):
            for j in range(TW // 16):
                nbuf[r, pl.ds(j * 16, 16)] = zeros16
            return 0
        lax.fori_loop(0, RCH, zsrc, 0)

        def zacc(i, _):
            cid = t + NT * i

            @pl.when(cid < NCH)
            def _():
                pltpu.sync_copy(nbuf, acc.at[pl.ds(cid * RCH, RCH)])
            return 0
        lax.fori_loop(0, (NCH + NT - 1) // NT, zacc, 0)
        plsc.subcore_barrier()

        par = [dict(zip(
            ["gidx", "sidx", "rows", "wrows",
             "semI", "semI2", "semG", "semS"],
            bufs[i * 8:(i + 1) * 8])) for i in range(2)]

        def issue_gidx(P, w):
            win = t * NWIN + w
            pltpu.async_copy(gsrc.at[sc, win], P["gidx"], P["semI"])

        def wait_gidx(P):
            pltpu.make_async_copy(
                gsrc.at[0, 0], P["gidx"], P["semI"]).wait()

        def issue_sidx(P, w):
            b = t * PAIRS_PER_TILE + w * W
            pltpu.async_copy(ssrc.at[pl.ds(b, W)], P["sidx"], P["semI2"])

        def wait_sidx(P):
            pltpu.make_async_copy(
                ssrc.at[pl.ds(0, W)], P["sidx"], P["semI2"]).wait()

        def issue_gather(P):
            pltpu.async_copy(table.at[P["gidx"].at[0]], P["rows"], P["semG"])

        def wait_gather(P):
            pltpu.make_async_copy(
                table.at[P["gidx"].at[0]], P["rows"], P["semG"]).wait()

        def issue_scatter(P):
            pltpu.async_copy(P["wrows"], acc.at[P["sidx"]], P["semS"],
                             add=True)

        def wait_scatter(P):
            pltpu.make_async_copy(P["wrows"], acc.at[P["sidx"]],
                                  P["semS"]).wait()

        def compute(P, w):
            rows, wrows = P["rows"], P["wrows"]

            def group(g, _):
                base16 = g * 16
                clsvec = P["gidx"][1, pl.ds(base16, 16)]
                for j in range(16):
                    p = base16 + j
                    cls = clsvec[j]
                    svec = rows[p, pl.ds(HH, 16)]
                    svec = jnp.where(svec > 0, svec, NEG * svec)
                    ev = jnp.exp(svec)
                    for h in range(4):
                        evh = vshuf(ev, jnp.full((16,), cls * 4 + h,
                                                 jnp.int32))
                        r = rows[p, pl.ds(h * 16, 16)]
                        wrows[p, pl.ds(h * 16, 16)] = r * evh
                    easm = vshuf(ev, cls * 4 + quad)
                    wrows[p, pl.ds(HH, 16)] = jnp.where(iota16 < 4, easm,
                                                        0.0)
                return 0
            lax.fori_loop(0, W // 16, group, 0)

        A, B = par
        issue_sidx(A, 0)
        issue_sidx(B, 1)
        issue_gidx(A, 0)
        issue_gidx(B, 1)
        wait_gidx(A)
        issue_gather(A)

        def pipe(i, _):
            w0 = 2 * i
            wait_gather(A)
            wait_gidx(B)
            issue_gather(B)
            compute(A, w0)
            issue_gidx(A, w0 + 2)
            wait_sidx(A)
            issue_scatter(A)

            w1 = w0 + 1

            @pl.when(i > 0)
            def _():
                wait_scatter(B)
                issue_sidx(B, w1)

            wait_gather(B)

            @pl.when(w1 + 2 <= NWIN)
            def _():
                wait_gidx(A)
                issue_gather(A)
            compute(B, w1)
            issue_gidx(B, w1 + 2)
            wait_sidx(B)
            issue_scatter(B)
            wait_scatter(A)
            issue_sidx(A, w0 + 2)
            return 0
        lax.fori_loop(0, NWIN // 2, pipe, 0)
        wait_gidx(A)
        wait_gidx(B)
        wait_sidx(A)
        wait_scatter(B)
        plsc.subcore_barrier()

        def chunk(i, _):
            cid = t + NT * i

            @pl.when(cid < NCH)
            def _():
                rbase = cid * RCH
                pltpu.sync_copy(acc.at[pl.ds(rbase, RCH)], nbuf)

                def row(r, _):
                    recd = 1.0 / (nbuf[r, pl.ds(HH, 16)] + 1e-16)
                    for h in range(4):
                        v = nbuf[r, pl.ds(h * 16, 16)] * vshuf(
                            recd, jnp.full((16,), h, jnp.int32))
                        if first:
                            v = jnp.maximum(v, 0.0)
                        nbuf[r, pl.ds(h * 16, 16)] = v
                    return 0
                lax.fori_loop(0, RCH, row, 0)
                if first:
                    pltpu.sync_copy(nbuf, out.at[pl.ds(sc * E + rbase, RCH)])
                else:
                    pltpu.sync_copy(nbuf, out.at[sc, pl.ds(rbase, RCH)])
            return 0
        lax.fori_loop(0, (NCH + NT - 1) // NT, chunk, 0)

    return body


_sc_round1 = _sc_round_kernel(True)
_sc_round2 = _sc_round_kernel(False)


def kernel(X, Wt, bt, att_e, att_v, vertex, edges, v_type,
           e_idx0, e_idx1, e_idx2, e_idx3, e_inv,
           v_idx0, v_idx1, v_idx2, v_idx3, v_inv):
    vertex = vertex.astype(jnp.int32)
    edges = edges.astype(jnp.int32)

    se0 = e_idx0.shape[0]
    se1 = se0 + e_idx1.shape[0]
    se2 = se1 + e_idx2.shape[0]
    cls_e = ((e_inv >= se0).astype(jnp.int32) + (e_inv >= se1)
             + (e_inv >= se2))
    sv0 = v_idx0.shape[0]
    sv1 = sv0 + v_idx1.shape[0]
    sv2 = sv1 + v_idx2.shape[0]
    cls_v = ((v_inv >= sv0).astype(jnp.int32) + (v_inv >= sv1)
             + (v_inv >= sv2))

    nwin = NNZ // W

    def pack_windows(gidx2, cls):
        g = gidx2.reshape(2, nwin, W)
        c = jnp.broadcast_to(cls[None], (2, NNZ)).reshape(2, nwin, W)
        comb = jnp.stack([g, c], axis=2)
        return jnp.concatenate(
            [comb, jnp.zeros((2, 2, 2, W), jnp.int32)], axis=1)

    vcomb = pack_windows(jnp.stack([vertex, vertex + N]), cls_e)
    ecomb = pack_windows(jnp.stack([edges, edges + E]), cls_v)
    spad = jnp.zeros((2 * W,), jnp.int32)
    edges_p = jnp.concatenate([edges, spad])
    vertex_p = jnp.concatenate([vertex, spad])

    ae = att_e.reshape(4, H, C)
    eye8 = jnp.eye(H, dtype=jnp.float32).reshape(H, 2, 4)
    me = (ae.transpose(1, 2, 0)[:, :, None, :, None]
          * eye8[:, None, :, None, :])
    me = me.reshape(D, 32)

    av = att_v.reshape(4, H, C).transpose(2, 0, 1).reshape(C, 4, 2, 4)
    eye4 = jnp.eye(4, dtype=jnp.float32)
    mv = (av.transpose(2, 3, 0, 1)[:, :, :, :, None]
          * eye4[None, :, None, None, :])
    mv = mv.reshape(2, HH, C)

    vtf = v_type.astype(jnp.float32).reshape(N, 1)
    xh = _typed_linear(X, vtf, Wt, bt, me)
    xh2 = xh.reshape(2 * N, TW)

    xe = _sc_round1(xh2, vcomb, edges_p)
    xe2 = _edge_scores(xe, mv)
    out = _sc_round2(xe2, ecomb, vertex_p)
    return out[:, :, :HH].transpose(1, 0, 2).reshape(N, H * C)
